# baseline (speedup 1.0000x reference)
import functools

import jax
import jax.numpy as jnp
from jax.experimental import pallas as pl
from jax.experimental.pallas import tpu as pltpu

F32 = jnp.float32
HI = jax.lax.Precision.HIGHEST


def _dot_nt(a, b, precision=HI):
    return jax.lax.dot_general(a, b, (((1,), (1,)), ((), ())),
                               precision=precision, preferred_element_type=F32)


def _dot_nn(a, b, precision=HI):
    return jax.lax.dot_general(a, b, (((1,), (0,)), ((), ())),
                               precision=precision, preferred_element_type=F32)


def _proj_body(x_ref, w_ref, b_ref, *out_refs, dout):
    xw = _dot_nt(x_ref[...], w_ref[...]) + b_ref[...]
    for j, oref in enumerate(out_refs):
        oref[...] = xw[:, j * dout:(j + 1) * dout]


def _proj(x, w, b, nsplit, bt=512):
    T, D = x.shape
    N = w.shape[0]
    dout = N // nsplit
    bt = min(bt, T)
    out_shape = tuple(jax.ShapeDtypeStruct((T, dout), F32) for _ in range(nsplit))
    outs = pl.pallas_call(
        functools.partial(_proj_body, dout=dout),
        grid=(T // bt,),
        in_specs=[
            pl.BlockSpec((bt, D), lambda i: (i, 0)),
            pl.BlockSpec((N, D), lambda i: (0, 0)),
            pl.BlockSpec((1, N), lambda i: (0, 0)),
        ],
        out_specs=tuple(pl.BlockSpec((bt, dout), lambda i: (i, 0))
                        for _ in range(nsplit)),
        out_shape=out_shape,
    )(x, w, b.reshape(1, N))
    return tuple(outs) if isinstance(outs, (list, tuple)) else (outs,)


def _attn_body(q_ref, k_ref, v_ref, o_ref, *, heads, dh, scale):
    q = q_ref[0]
    k = k_ref[0]
    v = v_ref[0]
    for h in range(heads):
        qh = q[:, h * dh:(h + 1) * dh]
        kh = k[:, h * dh:(h + 1) * dh]
        vh = v[:, h * dh:(h + 1) * dh]
        s = _dot_nt(qh, kh) * scale
        m = jnp.max(s, axis=-1, keepdims=True)
        p = jnp.exp(s - m)
        a = p / jnp.sum(p, axis=-1, keepdims=True)
        o_ref[0, :, h * dh:(h + 1) * dh] = _dot_nn(a, vh)


def _attn(q, k, v, heads, bq=512):
    B, S, D = q.shape
    M = k.shape[1]
    dh = D // heads
    bq = min(bq, S)
    return pl.pallas_call(
        functools.partial(_attn_body, heads=heads, dh=dh,
                          scale=1.0 / (float(dh) ** 0.5)),
        grid=(B, S // bq),
        in_specs=[
            pl.BlockSpec((1, bq, D), lambda b, i: (b, i, 0)),
            pl.BlockSpec((1, M, D), lambda b, i: (b, 0, 0)),
            pl.BlockSpec((1, M, D), lambda b, i: (b, 0, 0)),
        ],
        out_specs=pl.BlockSpec((1, bq, D), lambda b, i: (b, i, 0)),
        out_shape=jax.ShapeDtypeStruct((B, S, D), F32),
    )(q, k, v)


def _out_ln_body(ao_ref, w_ref, b_ref, res_ref, g_ref, be_ref, o_ref):
    y = _dot_nt(ao_ref[...], w_ref[...]) + b_ref[...] + res_ref[...]
    m = jnp.mean(y, axis=-1, keepdims=True)
    c = y - m
    var = jnp.mean(c * c, axis=-1, keepdims=True)
    o_ref[...] = c / jnp.sqrt(var + 1e-5) * g_ref[...] + be_ref[...]


def _out_ln(ao, wo, bo, res, g, be, bt=512):
    T, D = ao.shape
    bt = min(bt, T)
    return pl.pallas_call(
        _out_ln_body,
        grid=(T // bt,),
        in_specs=[
            pl.BlockSpec((bt, D), lambda i: (i, 0)),
            pl.BlockSpec((D, D), lambda i: (0, 0)),
            pl.BlockSpec((1, D), lambda i: (0, 0)),
            pl.BlockSpec((bt, D), lambda i: (i, 0)),
            pl.BlockSpec((1, D), lambda i: (0, 0)),
            pl.BlockSpec((1, D), lambda i: (0, 0)),
        ],
        out_specs=pl.BlockSpec((bt, D), lambda i: (i, 0)),
        out_shape=jax.ShapeDtypeStruct((T, D), F32),
    )(ao, wo, bo.reshape(1, D), res, g.reshape(1, D), be.reshape(1, D))


def _router_body(x_ref, rw_ref, rb_ref, noise_ref, g_ref, *, ne):
    lg = _dot_nt(x_ref[...], rw_ref[...]) + rb_ref[...]
    logits = lg[:, :ne]
    nl = lg[:, ne:]
    sp = jnp.maximum(nl, 0.0) + jnp.log(1.0 + jnp.exp(-jnp.abs(nl)))
    noisy = logits + noise_ref[...] * sp
    idx = jax.lax.broadcasted_iota(jnp.int32, noisy.shape, 1)
    m1 = jnp.max(noisy, axis=1, keepdims=True)
    i1 = jnp.min(jnp.where(noisy == m1, idx, ne), axis=1, keepdims=True)
    masked = jnp.where(idx == i1, -jnp.inf, noisy)
    m2 = jnp.max(masked, axis=1, keepdims=True)
    i2 = jnp.min(jnp.where(masked == m2, idx, ne), axis=1, keepdims=True)
    e2 = jnp.exp(m2 - m1)
    den = 1.0 + e2
    gates = jnp.where(idx == i1, 1.0 / den, jnp.where(idx == i2, e2 / den, 0.0))
    for e in range(ne):
        g_ref[e] = gates[:, e:e + 1]


def _router(x, rg_w, rg_b, rn_w, rn_b, noise, bt=512):
    T, D = x.shape
    E = rg_w.shape[0]
    bt = min(bt, T)
    rw = jnp.concatenate([rg_w, rn_w], axis=0)
    rb = jnp.concatenate([rg_b, rn_b], axis=0)
    return pl.pallas_call(
        functools.partial(_router_body, ne=E),
        grid=(T // bt,),
        in_specs=[
            pl.BlockSpec((bt, D), lambda i: (i, 0)),
            pl.BlockSpec((2 * E, D), lambda i: (0, 0)),
            pl.BlockSpec((1, 2 * E), lambda i: (0, 0)),
            pl.BlockSpec((bt, E), lambda i: (i, 0)),
        ],
        out_specs=pl.BlockSpec((E, bt, 1), lambda i: (0, i, 0)),
        out_shape=jax.ShapeDtypeStruct((E, T, 1), F32),
    )(x, rw, rb.reshape(1, 2 * E), noise)


def _moe_body(x_ref, g_ref, w1_ref, b1_ref, w2_ref, b2_ref, gam_ref, bet_ref,
              o_ref, *, ne):
    e = pl.program_id(1)
    xb = x_ref[...].astype(jnp.bfloat16)
    h = jnp.maximum(_dot_nt(xb, w1_ref[0], precision=None) + b1_ref[0], 0.0)
    eo = _dot_nt(h.astype(jnp.bfloat16), w2_ref[0], precision=None) + b2_ref[0]
    contrib = eo * g_ref[0]

    @pl.when(e == 0)
    def _():
        o_ref[...] = contrib

    @pl.when(e > 0)
    def _():
        o_ref[...] += contrib

    @pl.when(e == ne - 1)
    def _():
        y = o_ref[...] + x_ref[...]
        m = jnp.mean(y, axis=-1, keepdims=True)
        c = y - m
        var = jnp.mean(c * c, axis=-1, keepdims=True)
        o_ref[...] = c / jnp.sqrt(var + 1e-5) * gam_ref[...] + bet_ref[...]


def _moe_ln(x, gates, w1, b1, w2, b2, gam, bet, bt=1024):
    T, D = x.shape
    E, F, _ = w1.shape
    bt = min(bt, T)
    w1b = w1.astype(jnp.bfloat16)
    w2b = w2.astype(jnp.bfloat16)
    return pl.pallas_call(
        functools.partial(_moe_body, ne=E),
        grid=(T // bt, E),
        in_specs=[
            pl.BlockSpec((bt, D), lambda t, e: (t, 0)),
            pl.BlockSpec((1, bt, 1), lambda t, e: (e, t, 0)),
            pl.BlockSpec((1, F, D), lambda t, e: (e, 0, 0)),
            pl.BlockSpec((1, 1, F), lambda t, e: (e, 0, 0)),
            pl.BlockSpec((1, D, F), lambda t, e: (e, 0, 0)),
            pl.BlockSpec((1, 1, D), lambda t, e: (e, 0, 0)),
            pl.BlockSpec((1, D), lambda t, e: (0, 0)),
            pl.BlockSpec((1, D), lambda t, e: (0, 0)),
        ],
        out_specs=pl.BlockSpec((bt, D), lambda t, e: (t, 0)),
        out_shape=jax.ShapeDtypeStruct((T, D), F32),
    )(x, gates, w1b, b1.reshape(E, 1, F), w2b, b2.reshape(E, 1, D),
      gam.reshape(1, D), bet.reshape(1, D))


def kernel(tgt, memory, sa_w, sa_b, sa_wo, sa_bo, ma_w, ma_b, ma_wo, ma_bo,
           rg_w, rg_b, rn_w, rn_b, e_w1, e_b1, e_w2, e_b2,
           n1_g, n1_b, n2_g, n2_b, n3_g, n3_b, noise, heads=16):
    B, S, D = tgt.shape
    M = memory.shape[1]
    E = rg_w.shape[0]
    T = B * S

    x = tgt.reshape(T, D)
    q, k, v = _proj(x, sa_w, sa_b, 3)
    ao = _attn(q.reshape(B, S, D), k.reshape(B, S, D), v.reshape(B, S, D), heads)
    x = _out_ln(ao.reshape(T, D), sa_wo, sa_bo, x, n1_g, n1_b)

    mem = memory.reshape(B * M, D)
    (q2,) = _proj(x, ma_w[:D], ma_b[:D], 1)
    k2, v2 = _proj(mem, ma_w[D:], ma_b[D:], 2)
    ao2 = _attn(q2.reshape(B, S, D), k2.reshape(B, M, D), v2.reshape(B, M, D), heads)
    x = _out_ln(ao2.reshape(T, D), ma_wo, ma_bo, x, n2_g, n2_b)

    gates = _router(x, rg_w, rg_b, rn_w, rn_b, noise.reshape(T, E))
    out = _moe_ln(x, gates, e_w1, e_b1, e_w2, e_b2, n3_g, n3_b)
    return out.reshape(B, S, D)
